# Initial kernel scaffold; baseline (speedup 1.0000x reference)
import jax
import jax.numpy as jnp
from jax.experimental import pallas as pl


def kernel(tokens, edge_index, edge_attr, batch, emb_table, W_proj, b_proj, Ws1, Wm1, We1, Ws2, Wm2, We2, Ws3, Wm3, We3):
    raise NotImplementedError("write your pallas kernel here")


# speedup vs baseline: 2.1480x; 2.1480x over previous
import functools

import jax
import jax.numpy as jnp
from jax import lax
from jax.experimental import pallas as pl
from jax.experimental.pallas import tpu as pltpu
from jax.experimental.pallas import tpu_sc as plsc

N = 10000
E = 160000
VOCAB = 49412
D = 512
MID = 64
EP = 16
G = 64

NC, NS = 2, 16
NW = NC * NS

NP = 10240
EPAD = 163840
TPN = NP // NW
SLAB = NP // NS
EPT = EPAD // NW
CH = 128
EC = EPT // CH
GCH = 80
GC = TPN // GCH

_f32 = jnp.float32
_i32 = jnp.int32
_MESH = plsc.VectorSubcoreMesh(core_axis_name="c", subcore_axis_name="s")
_SC_PARAMS = pltpu.CompilerParams(use_tc_tiling_on_sc=False)


@functools.partial(
    pl.kernel,
    out_type=jax.ShapeDtypeStruct((NP, D), _f32),
    mesh=_MESH,
    compiler_params=_SC_PARAMS,
    scratch_types=[
        pltpu.VMEM((GC, GCH), _i32),
        pltpu.VMEM((GCH, D), _f32),
        pltpu.SemaphoreType.DMA,
    ],
)
def _embed(emb_hbm, tok_hbm, out_hbm, idx_v, rows_v, sem):
    c = lax.axis_index("c")
    s = lax.axis_index("s")
    wid = s * NC + c
    pltpu.sync_copy(tok_hbm.at[wid], idx_v)

    def body(j, _):
        rb = pl.multiple_of(wid * TPN + j * GCH, GCH)
        pltpu.async_copy(emb_hbm.at[idx_v.at[j]], rows_v, sem).wait()
        pltpu.sync_copy(rows_v, out_hbm.at[pl.ds(rb, GCH), :])
        return 0

    lax.fori_loop(0, GC, body, 0)


def _make_sc_pass(with_edges):
    out_type = [jax.ShapeDtypeStruct((NC, NP, MID), _f32)]
    scratch = [
        pltpu.VMEM((EC, CH), _i32),
        pltpu.VMEM((EC, CH), _i32),
        pltpu.VMEM((CH, MID), _f32),
        pltpu.VMEM((CH, MID), _f32),
        pltpu.VMEM_SHARED((NP, MID), _f32),
        pltpu.SemaphoreType.DMA,
    ]
    if with_edges:
        out_type += [
            jax.ShapeDtypeStruct((EPAD, EP), _f32),
            jax.ShapeDtypeStruct((EPAD, EP), _f32),
        ]
        scratch += [
            pltpu.VMEM((CH, EP), _f32),
            pltpu.VMEM((CH, EP), _f32),
        ]

    def body_fn(*refs):
        if with_edges:
            (p_hbm, q_hbm, u_hbm, v_hbm, src_hbm, dst_hbm, z_hbm,
             agg_out, ug_out, vg_out,
             src_v, dst_v, pbuf, qbuf, agg_sh, sem, ubuf, vbuf) = refs
        else:
            (p_hbm, q_hbm, src_hbm, dst_hbm, z_hbm,
             agg_out,
             src_v, dst_v, pbuf, qbuf, agg_sh, sem) = refs
        c = lax.axis_index("c")
        s = lax.axis_index("s")
        wid = s * NC + c
        slab = pl.multiple_of(s * SLAB, SLAB)
        pltpu.sync_copy(z_hbm.at[pl.ds(slab, SLAB), :],
                        agg_sh.at[pl.ds(slab, SLAB), :])
        pltpu.sync_copy(src_hbm.at[wid], src_v)
        pltpu.sync_copy(dst_hbm.at[wid], dst_v)
        plsc.subcore_barrier()

        def body(j, _):
            cb = pl.multiple_of(wid * EPT + j * CH, CH)
            pltpu.async_copy(p_hbm.at[src_v.at[j]], pbuf, sem).wait()
            pltpu.sync_copy(q_hbm.at[pl.ds(cb, CH), :], qbuf)
            pltpu.sync_copy(pbuf, agg_sh.at[dst_v.at[j]], add=True)
            pltpu.sync_copy(qbuf, agg_sh.at[dst_v.at[j]], add=True)
            if with_edges:
                pltpu.async_copy(u_hbm.at[src_v.at[j]], ubuf, sem).wait()
                pltpu.sync_copy(ubuf, ug_out.at[pl.ds(cb, CH), :])
                pltpu.async_copy(v_hbm.at[dst_v.at[j]], vbuf, sem).wait()
                pltpu.sync_copy(vbuf, vg_out.at[pl.ds(cb, CH), :])
            return 0

        lax.fori_loop(0, EC, body, 0)
        plsc.subcore_barrier()
        pltpu.sync_copy(agg_sh.at[pl.ds(slab, SLAB), :],
                        agg_out.at[c, pl.ds(slab, SLAB), :])

    return pl.kernel(body_fn, out_type=out_type, mesh=_MESH,
                     compiler_params=_SC_PARAMS, scratch_types=scratch)


_sc_pass_edges = _make_sc_pass(True)
_sc_pass_agg = _make_sc_pass(False)


def _mm_body(n_in, relu, n_w):
    def body(*refs):
        ins = refs[:n_in]
        wrefs = refs[n_in:n_in + n_w]
        outs = refs[n_in + n_w:]
        x = ins[0][...]
        for r in ins[1:]:
            x = x + r[...]
        if relu:
            x = jnp.maximum(x, 0.0)
        for w_ref, o_ref in zip(wrefs, outs):
            o_ref[...] = jnp.dot(x, w_ref[...],
                                 preferred_element_type=jnp.float32)
    return body


def _mm(xs, ws, relu, bm):
    m, k = xs[0].shape
    in_specs = (
        [pl.BlockSpec((bm, k), lambda i: (i, 0)) for _ in xs]
        + [pl.BlockSpec(w.shape, lambda i: (0, 0)) for w in ws]
    )
    out = pl.pallas_call(
        _mm_body(len(xs), relu, len(ws)),
        grid=(m // bm,),
        in_specs=in_specs,
        out_specs=[pl.BlockSpec((bm, w.shape[1]), lambda i: (i, 0))
                   for w in ws],
        out_shape=[jax.ShapeDtypeStruct((m, w.shape[1]), _f32) for w in ws],
    )(*xs, *ws)
    return out


def _edge1_body(ea_ref, wp_ref, bp_ref, wq_ref, ww_ref, q_ref, w_ref):
    e = jnp.dot(ea_ref[...], wp_ref[...],
                preferred_element_type=jnp.float32) + bp_ref[...]
    q_ref[...] = jnp.dot(e, wq_ref[...], preferred_element_type=jnp.float32)
    w_ref[...] = jnp.dot(e, ww_ref[...], preferred_element_type=jnp.float32)


def _edge1(ea, wp, bp, wq, ww, bm):
    m, k = ea.shape
    return pl.pallas_call(
        _edge1_body,
        grid=(m // bm,),
        in_specs=[
            pl.BlockSpec((bm, k), lambda i: (i, 0)),
            pl.BlockSpec(wp.shape, lambda i: (0, 0)),
            pl.BlockSpec(bp.shape, lambda i: (0, 0)),
            pl.BlockSpec(wq.shape, lambda i: (0, 0)),
            pl.BlockSpec(ww.shape, lambda i: (0, 0)),
        ],
        out_specs=[
            pl.BlockSpec((bm, wq.shape[1]), lambda i: (i, 0)),
            pl.BlockSpec((bm, ww.shape[1]), lambda i: (i, 0)),
        ],
        out_shape=[
            jax.ShapeDtypeStruct((m, wq.shape[1]), _f32),
            jax.ShapeDtypeStruct((m, ww.shape[1]), _f32),
        ],
    )(ea, wp, bp, wq, ww)


def _final_body(a_ref, b_ref, c_ref, batch_ref, o_ref):
    x3 = a_ref[...] + b_ref[...] + c_ref[...]
    bt = batch_ref[...]
    ids = lax.broadcasted_iota(_i32, (G, NP), 1)
    gid = lax.broadcasted_iota(_i32, (G, NP), 0)
    cand = jnp.where(bt == gid, ids, NP)
    fi = jnp.min(cand, axis=1, keepdims=True)
    fi = jnp.minimum(fi, N - 1)
    sel = (ids == fi).astype(jnp.float32)
    o_ref[...] = jnp.dot(sel, x3, preferred_element_type=jnp.float32)


def _final(a, b, c, batch2):
    return pl.pallas_call(
        _final_body,
        in_specs=[
            pl.BlockSpec(a.shape, lambda: (0, 0)),
            pl.BlockSpec(b.shape, lambda: (0, 0)),
            pl.BlockSpec(c.shape, lambda: (0, 0)),
            pl.BlockSpec(batch2.shape, lambda: (0, 0)),
        ],
        out_specs=pl.BlockSpec((G, MID), lambda: (0, 0)),
        out_shape=jax.ShapeDtypeStruct((G, MID), _f32),
    )(a, b, c, batch2)


def kernel(tokens, edge_index, edge_attr, batch, emb_table, W_proj, b_proj,
           Ws1, Wm1, We1, Ws2, Wm2, We2, Ws3, Wm3, We3):
    tok_p = jnp.concatenate(
        [tokens.astype(_i32), jnp.zeros((NP - N,), _i32)]
    ).reshape(NW, GC, GCH)
    npad = EPAD - E
    pad_idx = N + (jnp.arange(npad, dtype=_i32) % (NP - N))
    src_p = jnp.concatenate([edge_index[0].astype(_i32), pad_idx]
                            ).reshape(NW, EC, CH)
    dst_p = jnp.concatenate([edge_index[1].astype(_i32), pad_idx]
                            ).reshape(NW, EC, CH)
    ea_p = jnp.concatenate(
        [edge_attr, jnp.zeros((npad, edge_attr.shape[1]), _f32)])
    batch2 = jnp.concatenate(
        [batch.astype(_i32), jnp.full((NP - N,), G, _i32)]).reshape(1, NP)
    zeros_n = jnp.zeros((NP, MID), _f32)

    x0 = _embed(emb_table, tok_p)

    p1, s1, u1, v1 = _mm([x0], [Wm1[:D], Ws1, We1[:D], We1[D:2 * D]],
                         relu=False, bm=1024)
    q1, w1 = _edge1(ea_p, W_proj, b_proj.reshape(1, EP),
                    Wm1[D:], We1[2 * D:], bm=8192)
    agg1, ug1, vg1 = _sc_pass_edges(p1, q1, u1, v1, src_p, dst_p, zeros_n)

    p2, s2, u2, v2 = _mm([s1, agg1[0], agg1[1]],
                         [Wm2[:MID], Ws2, We2[:MID], We2[MID:2 * MID]],
                         relu=True, bm=2048)
    q2, w2 = _mm([ug1, vg1, w1], [Wm2[MID:], We2[2 * MID:]],
                 relu=True, bm=8192)
    agg2, ug2, vg2 = _sc_pass_edges(p2, q2, u2, v2, src_p, dst_p, zeros_n)

    p3, s3 = _mm([s2, agg2[0], agg2[1]], [Wm3[:MID], Ws3],
                 relu=True, bm=2048)
    (q3,) = _mm([ug2, vg2, w2], [Wm3[MID:]], relu=True, bm=8192)
    agg3 = _sc_pass_agg(p3, q3, src_p, dst_p, zeros_n)

    return _final(s3, agg3[0][0], agg3[0][1], batch2)
